# baseline (speedup 1.0000x reference)
import functools

import jax
import jax.numpy as jnp
from jax.experimental import pallas as pl

_M = 10
_NF = 256
_P = 256
_K = 0.01
_T = _M // 2
_HI = jax.lax.Precision.HIGHEST
_LO = jax.lax.Precision.DEFAULT


def _dg(row, mat, precision=_HI):
    return jax.lax.dot_general(
        row, mat, (((1,), (1,)), ((), ())),
        precision=precision, preferred_element_type=jnp.float32)


def _s2v_kernel(z_ref, xp_ref, b1_ref, th1_ref, th2_ref, th3_ref, th2c_ref,
                th3c_ref, th4a_ref, th4b_ref, mu0_ref, out_ref):
    z = z_ref[0, 0]
    b1 = b1_ref[0, 0]
    b2 = b1 * b1

    xp = xp_ref[...]
    gram = _dg(xp, xp)

    rid = jax.lax.broadcasted_iota(jnp.int32, (16, 16), 0)
    cid = jax.lax.broadcasted_iota(jnp.int32, (16, 16), 1)
    nb_mask = (cid < _M) & (cid != rid)
    w = gram + b2
    pos = jnp.sum(jnp.where(nb_mask, jnp.maximum(w, 0.0), 0.0), axis=1,
                  keepdims=True)
    neg = jnp.sum(jnp.where(nb_mask, jnp.maximum(-w, 0.0), 0.0), axis=1,
                  keepdims=True)

    g9 = jnp.sum(jnp.where((rid == _M) & (cid == 9), gram, 0.0))
    g8 = jnp.sum(jnp.where((rid == _M) & (cid == 8), gram, 0.0))
    vcol = jax.lax.broadcasted_iota(jnp.int32, (16, 1), 0)
    wlast = jnp.where(vcol == 9, g8, g9)

    arows = jax.lax.broadcasted_iota(jnp.int32, (16, 1), 0)
    a_z = jnp.sum(jnp.where(arows == z, xp, 0.0), axis=0, keepdims=True)
    t1 = _K * _dg(a_z, th1_ref[...])

    th3 = th3_ref[...]
    th3c = th3c_ref[...]
    base = (t1
            + _K * (pos * jnp.maximum(th3, 0.0)
                    + neg * jnp.maximum(-th3, 0.0))
            + _K * (jnp.maximum(wlast, 0.0) * jnp.maximum(th3c, 0.0)
                    + jnp.maximum(-wlast, 0.0) * jnp.maximum(-th3c, 0.0)))

    th2 = th2_ref[...]
    th2a = th2[:, :128]
    th2b = th2[:, 128:]
    th2c = th2c_ref[...]

    def t2mv(u):
        return (_dg(u[:, :128], th2a, _LO) + _dg(u[:, 128:], th2b, _LO))
    cols = [mu0_ref[v:v + 1, :] for v in range(_M + 1)]
    s_all = functools.reduce(jnp.add, cols[:_M])
    for _ in range(_T):
        bc = base + _K * _dg(cols[_M], th2c)
        sv = s_all - cols[0]
        for v in range(_M):
            x = jnp.maximum(bc[v:v + 1, :] + _K * t2mv(sv), 0.0)
            s_all = sv + x
            cols[v] = x
            if v + 1 < _M:
                sv = s_all - cols[v + 1]
        cols[_M] = jnp.maximum(bc[_M:_M + 1, :] + _K * t2mv(s_all), 0.0)

    r4a = _dg(s_all, th4a_ref[...])
    mu_z = functools.reduce(
        jnp.add,
        [jnp.where(z == v, 1.0, 0.0) * cols[v] for v in range(_M)])
    r4b = _dg(mu_z, th4b_ref[...])
    out_ref[...] = jnp.concatenate([r4a, r4b], axis=1)


@jax.jit
def kernel(A, b, c, z, theta1, theta2, theta3, theta2c, theta3c, theta4a,
           theta4b):
    a0 = A[0]
    xp = jnp.concatenate(
        [a0, c, jnp.zeros((5, _NF), jnp.float32)], axis=0)
    b1 = b[:, 1, 0].reshape(1, 1)
    zz = z.reshape(1, 1)
    mu0 = jax.random.normal(jax.random.key(42), (_P, _M + 1),
                            dtype=jnp.float32).T
    out = pl.pallas_call(
        _s2v_kernel,
        out_shape=jax.ShapeDtypeStruct((1, 2 * _P), jnp.float32),
    )(zz, xp, b1, theta1, theta2, theta3.reshape(1, _P),
      theta2c, theta3c.reshape(1, _P), theta4a, theta4b, mu0)
    return out


# speedup vs baseline: 1.1207x; 1.1207x over previous
import functools

import jax
import jax.numpy as jnp
from jax.experimental import pallas as pl

_M = 10
_NF = 256
_P = 256
_K = 0.01
_T = _M // 2
_HI = jax.lax.Precision.HIGHEST
_LO = jax.lax.Precision.DEFAULT


def _dg(row, mat, precision=_HI):
    return jax.lax.dot_general(
        row, mat, (((1,), (1,)), ((), ())),
        precision=precision, preferred_element_type=jnp.float32)


def _s2v_kernel(z_ref, xp_ref, b1_ref, th1_ref, th2_ref, th3_ref, th2c_ref,
                th3c_ref, th4a_ref, th4b_ref, mu0_ref, out_ref):
    z = z_ref[0, 0]
    b1 = b1_ref[0, 0]
    b2 = b1 * b1

    xp = xp_ref[...]
    gram = _dg(xp, xp)

    rid = jax.lax.broadcasted_iota(jnp.int32, (16, 16), 0)
    cid = jax.lax.broadcasted_iota(jnp.int32, (16, 16), 1)
    nb_mask = (cid < _M) & (cid != rid)
    w = gram + b2
    pos = jnp.sum(jnp.where(nb_mask, jnp.maximum(w, 0.0), 0.0), axis=1,
                  keepdims=True)
    neg = jnp.sum(jnp.where(nb_mask, jnp.maximum(-w, 0.0), 0.0), axis=1,
                  keepdims=True)

    g9 = jnp.sum(jnp.where((rid == _M) & (cid == 9), gram, 0.0))
    g8 = jnp.sum(jnp.where((rid == _M) & (cid == 8), gram, 0.0))
    vcol = jax.lax.broadcasted_iota(jnp.int32, (16, 1), 0)
    wlast = jnp.where(vcol == 9, g8, g9)

    arows = jax.lax.broadcasted_iota(jnp.int32, (16, 1), 0)
    a_z = jnp.sum(jnp.where(arows == z, xp, 0.0), axis=0, keepdims=True)
    t1 = _K * _dg(a_z, th1_ref[...])

    th3 = th3_ref[...]
    th3c = th3c_ref[...]
    base = (t1
            + _K * (pos * jnp.maximum(th3, 0.0)
                    + neg * jnp.maximum(-th3, 0.0))
            + _K * (jnp.maximum(wlast, 0.0) * jnp.maximum(th3c, 0.0)
                    + jnp.maximum(-wlast, 0.0) * jnp.maximum(-th3c, 0.0)))

    th2t = jnp.transpose(th2_ref[...])
    th2c = th2c_ref[...]

    def t2mv(u):
        u_col = jnp.reshape(u, (_P, 1))
        return _K * jnp.sum(th2t * u_col, axis=0, keepdims=True)
    cols = [mu0_ref[v:v + 1, :] for v in range(_M + 1)]
    s_all = functools.reduce(jnp.add, cols[:_M])
    for _ in range(_T):
        bc = base + _K * _dg(cols[_M], th2c)
        sv = s_all - cols[0]
        for v in range(_M):
            x = jnp.maximum(bc[v:v + 1, :] + t2mv(sv), 0.0)
            s_all = sv + x
            cols[v] = x
            if v + 1 < _M:
                sv = s_all - cols[v + 1]
        cols[_M] = jnp.maximum(bc[_M:_M + 1, :] + t2mv(s_all), 0.0)

    r4a = _dg(s_all, th4a_ref[...])
    mu_z = functools.reduce(
        jnp.add,
        [jnp.where(z == v, 1.0, 0.0) * cols[v] for v in range(_M)])
    r4b = _dg(mu_z, th4b_ref[...])
    out_ref[...] = jnp.concatenate([r4a, r4b], axis=1)


@jax.jit
def kernel(A, b, c, z, theta1, theta2, theta3, theta2c, theta3c, theta4a,
           theta4b):
    a0 = A[0]
    xp = jnp.concatenate(
        [a0, c, jnp.zeros((5, _NF), jnp.float32)], axis=0)
    b1 = b[:, 1, 0].reshape(1, 1)
    zz = z.reshape(1, 1)
    mu0 = jax.random.normal(jax.random.key(42), (_P, _M + 1),
                            dtype=jnp.float32).T
    out = pl.pallas_call(
        _s2v_kernel,
        out_shape=jax.ShapeDtypeStruct((1, 2 * _P), jnp.float32),
    )(zz, xp, b1, theta1, theta2, theta3.reshape(1, _P),
      theta2c, theta3c.reshape(1, _P), theta4a, theta4b, mu0)
    return out


# speedup vs baseline: 1.1659x; 1.0403x over previous
import functools

import jax
import jax.numpy as jnp
import numpy as np
from jax.experimental import pallas as pl

_M = 10
_NF = 256
_P = 256
_K = 0.01
_T = _M // 2
_HI = jax.lax.Precision.HIGHEST
_LO = jax.lax.Precision.DEFAULT

_MU0 = np.asarray(
    jax.random.normal(jax.random.key(42), (_P, _M + 1), dtype=jnp.float32),
    dtype=np.float32).T.copy()


def _dg(row, mat, precision=_HI):
    return jax.lax.dot_general(
        row, mat, (((1,), (1,)), ((), ())),
        precision=precision, preferred_element_type=jnp.float32)


def _s2v_kernel(z_ref, xp_ref, b1_ref, th1_ref, th2_ref, th3_ref, th2c_ref,
                th3c_ref, th4a_ref, th4b_ref, mu0_ref, out_ref):
    z = z_ref[0, 0]
    b1 = b1_ref[0, 0]
    b2 = b1 * b1

    xp = xp_ref[...]
    gram = _dg(xp, xp)

    rid = jax.lax.broadcasted_iota(jnp.int32, (16, 16), 0)
    cid = jax.lax.broadcasted_iota(jnp.int32, (16, 16), 1)
    nb_mask = (cid < _M) & (cid != rid)
    w = gram + b2
    pos = jnp.sum(jnp.where(nb_mask, jnp.maximum(w, 0.0), 0.0), axis=1,
                  keepdims=True)
    neg = jnp.sum(jnp.where(nb_mask, jnp.maximum(-w, 0.0), 0.0), axis=1,
                  keepdims=True)

    g9 = jnp.sum(jnp.where((rid == _M) & (cid == 9), gram, 0.0))
    g8 = jnp.sum(jnp.where((rid == _M) & (cid == 8), gram, 0.0))
    vcol = jax.lax.broadcasted_iota(jnp.int32, (16, 1), 0)
    wlast = jnp.where(vcol == 9, g8, g9)

    arows = jax.lax.broadcasted_iota(jnp.int32, (16, 1), 0)
    a_z = jnp.sum(jnp.where(arows == z, xp, 0.0), axis=0, keepdims=True)
    t1 = _K * _dg(a_z, th1_ref[...])

    th3 = th3_ref[...]
    th3c = th3c_ref[...]
    base = (t1
            + _K * (pos * jnp.maximum(th3, 0.0)
                    + neg * jnp.maximum(-th3, 0.0))
            + _K * (jnp.maximum(wlast, 0.0) * jnp.maximum(th3c, 0.0)
                    + jnp.maximum(-wlast, 0.0) * jnp.maximum(-th3c, 0.0)))

    th2t = jnp.transpose(th2_ref[...])
    th2c = th2c_ref[...]

    def t2mv(u):
        u_col = jnp.reshape(u, (_P, 1))
        return _K * jnp.sum(th2t * u_col, axis=0, keepdims=True)
    cols = [mu0_ref[v:v + 1, :] for v in range(_M + 1)]
    s_all = functools.reduce(jnp.add, cols[:_M])
    for _ in range(_T):
        bc = base + _K * _dg(cols[_M], th2c)
        sv = s_all - cols[0]
        for v in range(_M):
            x = jnp.maximum(bc[v:v + 1, :] + t2mv(sv), 0.0)
            s_all = sv + x
            cols[v] = x
            if v + 1 < _M:
                sv = s_all - cols[v + 1]
        cols[_M] = jnp.maximum(bc[_M:_M + 1, :] + t2mv(s_all), 0.0)

    r4a = _dg(s_all, th4a_ref[...])
    mu_z = functools.reduce(
        jnp.add,
        [jnp.where(z == v, 1.0, 0.0) * cols[v] for v in range(_M)])
    r4b = _dg(mu_z, th4b_ref[...])
    out_ref[...] = jnp.concatenate([r4a, r4b], axis=1)


@jax.jit
def kernel(A, b, c, z, theta1, theta2, theta3, theta2c, theta3c, theta4a,
           theta4b):
    a0 = A[0]
    xp = jnp.concatenate(
        [a0, c, jnp.zeros((5, _NF), jnp.float32)], axis=0)
    b1 = b[:, 1, 0].reshape(1, 1)
    zz = z.reshape(1, 1)
    mu0 = jnp.asarray(_MU0)
    out = pl.pallas_call(
        _s2v_kernel,
        out_shape=jax.ShapeDtypeStruct((1, 2 * _P), jnp.float32),
    )(zz, xp, b1, theta1, theta2, theta3.reshape(1, _P),
      theta2c, theta3c.reshape(1, _P), theta4a, theta4b, mu0)
    return out
